# baseline (speedup 1.0000x reference)
import jax
import jax.numpy as jnp
from jax import lax
from jax.experimental import pallas as pl
from jax.experimental.pallas import tpu as pltpu
from jax.experimental.pallas import tpu_sc as plsc

VOCAB = 100000
EMBED = 512
BATCH = 4096
SEQ = 30

NC = 2
NS = 16
NW = NC * NS

B = BATCH * SEQ
NB_PER_W = BATCH // NW

CHUNK = 64
NBUF = 3
GDEPTH = 2
HALVES = NB_PER_W // CHUNK
NCHUNKS = SEQ * HALVES
NGROUPS = NCHUNKS // NBUF
assert NB_PER_W % CHUNK == 0 and NCHUNKS % NBUF == 0 and GDEPTH < NBUF


def _gather_body(xt_hbm, table_hbm, out_hbm, xt_v, *rest):
    bufs = rest[:NBUF]
    gsems = rest[NBUF:2 * NBUF]
    ssems = rest[2 * NBUF:]

    wid = lax.axis_index("s") * NC + lax.axis_index("c")
    out_base = wid * NB_PER_W

    pltpu.sync_copy(xt_hbm.at[:, pl.ds(out_base, NB_PER_W)], xt_v)

    def idx_chunk(c):
        return xt_v.at[c // HALVES, pl.ds((c % HALVES) * CHUNK, CHUNK)]

    def start_gather(c, b):
        pltpu.async_copy(table_hbm.at[idx_chunk(c)], bufs[b], gsems[b])

    def wait_gather(b):
        pltpu.make_async_copy(table_hbm.at[idx_chunk(0)], bufs[b],
                              gsems[b]).wait()

    def start_scatter(c, b):
        s = c // HALVES
        h = c % HALVES
        row = s * BATCH + out_base + h * CHUNK
        pltpu.async_copy(bufs[b], out_hbm.at[pl.ds(row, CHUNK)], ssems[b])

    def wait_scatter(b):
        dst = out_hbm.at[pl.ds(out_base, CHUNK)]
        pltpu.make_async_copy(bufs[b], dst, ssems[b]).wait()

    for c0 in range(GDEPTH):
        start_gather(c0, c0 % NBUF)

    def group(g, _):
        for b in range(NBUF):
            c = g * NBUF + b
            wait_gather(b)
            start_scatter(c, b)
            cg = c + GDEPTH
            bg = (b + GDEPTH) % NBUF

            @pl.when(cg < NCHUNKS)
            def _():
                @pl.when(cg >= NBUF)
                def _():
                    wait_scatter(bg)

                start_gather(cg, bg)
        return 0

    lax.fori_loop(0, NGROUPS, group, 0)

    for b in range(NBUF):
        wait_scatter(b)


@jax.jit
def _gather(xt, table):
    mesh = plsc.VectorSubcoreMesh(
        core_axis_name="c", subcore_axis_name="s", num_cores=NC, num_subcores=NS
    )
    return pl.kernel(
        _gather_body,
        out_type=jax.ShapeDtypeStruct((B, EMBED), jnp.float32),
        mesh=mesh,
        compiler_params=pltpu.CompilerParams(needs_layout_passes=False),
        scratch_types=[
            pltpu.VMEM((SEQ, NB_PER_W), jnp.int32),
        ]
        + [pltpu.VMEM((CHUNK, EMBED), jnp.float32) for _ in range(NBUF)]
        + [pltpu.SemaphoreType.DMA for _ in range(2 * NBUF)],
    )(xt, table)


def kernel(x, table):
    out = _gather(x.T, table)
    return out.reshape(SEQ, BATCH, EMBED).transpose(1, 0, 2)
